# baseline (speedup 1.0000x reference)
import jax
import jax.numpy as jnp
from jax import lax
from jax.experimental import pallas as pl
from jax.experimental.pallas import tpu as pltpu
from jax.experimental.pallas import tpu_sc as plsc

_NC = 2
_NS = 16
_NW = _NC * _NS
_L = 16
_D = 32


def _mf_body(uidx_hbm, iidx_hbm, utab_hbm, itab_hbm, out_hbm,
             uidx_v, iidx_v, urows_v, irows_v, q_v, out_v, sem_u, sem_i):
    b_per_w = uidx_v.shape[0]
    wid = lax.axis_index("s") * _NC + lax.axis_index("c")
    base = wid * b_per_w

    pltpu.sync_copy(uidx_hbm.at[pl.ds(base, b_per_w)], uidx_v)
    pltpu.sync_copy(iidx_hbm.at[pl.ds(base, b_per_w)], iidx_v)
    cp_u = pltpu.async_copy(utab_hbm.at[uidx_v], urows_v, sem_u)
    cp_i = pltpu.async_copy(itab_hbm.at[iidx_v], irows_v, sem_i)
    cp_u.wait()
    cp_i.wait()

    lanes = lax.iota(jnp.int32, _L)

    def chunk_body(c, carry):
        b0 = c * _L
        for k in range(_L):
            u0 = urows_v[b0 + k, 0:16]
            u1 = urows_v[b0 + k, 16:32]
            i0 = irows_v[b0 + k, 0:16]
            i1 = irows_v[b0 + k, 16:32]
            q_v[pl.ds(k * (_L + 1), _L)] = u0 * i0 + u1 * i1
        acc = jnp.zeros((_L,), jnp.float32)
        stride_lanes = lanes * (_L + 1)
        for j in range(_L):
            acc = acc + plsc.load_gather(q_v, [stride_lanes + j])
        out_v[pl.ds(b0, _L)] = 1.0 / (1.0 + jnp.exp(-acc))
        return carry

    lax.fori_loop(0, b_per_w // _L, chunk_body, 0)
    pltpu.sync_copy(out_v, out_hbm.at[pl.ds(base, b_per_w)])


def kernel(user_indices, item_indices, user_table, item_table):
    B = user_indices.shape[0]
    assert B % (_NW * _L) == 0
    b_per_w = B // _NW
    mesh = plsc.VectorSubcoreMesh(core_axis_name="c", subcore_axis_name="s",
                                  num_cores=_NC, num_subcores=_NS)
    run = pl.kernel(
        _mf_body,
        out_type=jax.ShapeDtypeStruct((B,), jnp.float32),
        mesh=mesh,
        compiler_params=pltpu.CompilerParams(needs_layout_passes=False,
                                             use_tc_tiling_on_sc=False),
        scratch_types=[
            pltpu.VMEM((b_per_w,), jnp.int32),
            pltpu.VMEM((b_per_w,), jnp.int32),
            pltpu.VMEM((b_per_w, _D), jnp.float32),
            pltpu.VMEM((b_per_w, _D), jnp.float32),
            pltpu.VMEM((_L * (_L + 1),), jnp.float32),
            pltpu.VMEM((b_per_w,), jnp.float32),
            pltpu.SemaphoreType.DMA,
            pltpu.SemaphoreType.DMA,
        ],
    )
    return run(user_indices, item_indices, user_table, item_table)
